# Initial kernel scaffold; baseline (speedup 1.0000x reference)
import jax
import jax.numpy as jnp
from jax.experimental import pallas as pl


def kernel(x, edge_index, W_self1, W_neigh1, b1, W_self2, W_neigh2, b2):
    raise NotImplementedError("write your pallas kernel here")


# speedup vs baseline: 10.7189x; 10.7189x over previous
import functools

import jax
import jax.numpy as jnp
from jax import lax
from jax.experimental import pallas as pl
from jax.experimental.pallas import tpu as pltpu
from jax.experimental.pallas import tpu_sc as plsc

D = 128
CH = 128
ROWBLK = 1280


def _mm2_body(x_ref, wa_ref, wb_ref, oa_ref, ob_ref):
    x = x_ref[...]
    oa_ref[...] = jnp.dot(x, wa_ref[...], preferred_element_type=jnp.float32)
    ob_ref[...] = jnp.dot(x, wb_ref[...], preferred_element_type=jnp.float32)


def _tc_dual_matmul(x, wa, wb):
    n = x.shape[0]
    grid = (pl.cdiv(n, ROWBLK),)
    return pl.pallas_call(
        _mm2_body,
        grid=grid,
        in_specs=[
            pl.BlockSpec((ROWBLK, D), lambda i: (i, 0)),
            pl.BlockSpec((D, D), lambda i: (0, 0)),
            pl.BlockSpec((D, D), lambda i: (0, 0)),
        ],
        out_specs=[
            pl.BlockSpec((ROWBLK, D), lambda i: (i, 0)),
            pl.BlockSpec((ROWBLK, D), lambda i: (i, 0)),
        ],
        out_shape=[
            jax.ShapeDtypeStruct((n, D), jnp.float32),
            jax.ShapeDtypeStruct((n, D), jnp.float32),
        ],
    )(x, wa, wb)


def _mid_body(s1_ref, a0_ref, a1_ref, degt_ref, b1_ref, ws2_ref, wn2_ref,
              s2_ref, y2_ref):
    deg = jnp.sum(degt_ref[...], axis=1, keepdims=True)
    rdeg = 1.0 / jnp.maximum(deg, 1.0)
    agg = a0_ref[0] + a1_ref[0]
    h = jnp.maximum(s1_ref[...] + agg * rdeg + b1_ref[...], 0.0)
    s2_ref[...] = jnp.dot(h, ws2_ref[...], preferred_element_type=jnp.float32)
    y2_ref[...] = jnp.dot(h, wn2_ref[...], preferred_element_type=jnp.float32)


def _tc_mid(s1, aggp, degt, b1, ws2, wn2):
    n = s1.shape[0]
    grid = (pl.cdiv(n, ROWBLK),)
    return pl.pallas_call(
        _mid_body,
        grid=grid,
        in_specs=[
            pl.BlockSpec((ROWBLK, D), lambda i: (i, 0)),
            pl.BlockSpec((1, ROWBLK, D), lambda i: (0, i, 0)),
            pl.BlockSpec((1, ROWBLK, D), lambda i: (1, i, 0)),
            pl.BlockSpec((ROWBLK, degt.shape[1]), lambda i: (i, 0)),
            pl.BlockSpec((1, D), lambda i: (0, 0)),
            pl.BlockSpec((D, D), lambda i: (0, 0)),
            pl.BlockSpec((D, D), lambda i: (0, 0)),
        ],
        out_specs=[
            pl.BlockSpec((ROWBLK, D), lambda i: (i, 0)),
            pl.BlockSpec((ROWBLK, D), lambda i: (i, 0)),
        ],
        out_shape=[
            jax.ShapeDtypeStruct((n, D), jnp.float32),
            jax.ShapeDtypeStruct((n, D), jnp.float32),
        ],
    )(s1, aggp, aggp, degt, b1, ws2, wn2)


def _fin_body(s2_ref, a0_ref, a1_ref, degt_ref, b2_ref, out_ref):
    deg = jnp.sum(degt_ref[...], axis=1, keepdims=True)
    rdeg = 1.0 / jnp.maximum(deg, 1.0)
    agg = a0_ref[0] + a1_ref[0]
    out_ref[...] = s2_ref[...] + agg * rdeg + b2_ref[...]


def _tc_fin(s2, aggp, degt, b2):
    n = s2.shape[0]
    grid = (pl.cdiv(n, ROWBLK),)
    return pl.pallas_call(
        _fin_body,
        grid=grid,
        in_specs=[
            pl.BlockSpec((ROWBLK, D), lambda i: (i, 0)),
            pl.BlockSpec((1, ROWBLK, D), lambda i: (0, i, 0)),
            pl.BlockSpec((1, ROWBLK, D), lambda i: (1, i, 0)),
            pl.BlockSpec((ROWBLK, degt.shape[1]), lambda i: (i, 0)),
            pl.BlockSpec((1, D), lambda i: (0, 0)),
        ],
        out_specs=pl.BlockSpec((ROWBLK, D), lambda i: (i, 0)),
        out_shape=jax.ShapeDtypeStruct((n, D), jnp.float32),
    )(s2, aggp, aggp, degt, b2)


def _make_sc_agg(npad, nchunk, with_deg, nc, ns):
    nw = nc * ns
    rows_per_tile = npad // ns
    mesh = plsc.VectorSubcoreMesh(
        core_axis_name="c", subcore_axis_name="s", num_cores=nc, num_subcores=ns
    )

    out_type = [jax.ShapeDtypeStruct((2, npad, D), jnp.float32)]
    scratch = [
        pltpu.VMEM((2, CH), jnp.int32),
        pltpu.VMEM((2, CH), jnp.int32),
        pltpu.VMEM((CH, D), jnp.float32),
        pltpu.VMEM((CH, D), jnp.float32),
        pltpu.VMEM_SHARED((npad, D), jnp.float32),
        pltpu.SemaphoreType.DMA,
        pltpu.SemaphoreType.DMA,
    ]
    if with_deg:
        out_type.append(jax.ShapeDtypeStruct((nw, npad), jnp.float32))
        scratch.append(pltpu.VMEM((npad,), jnp.float32))

    def body(y_hbm, ed_hbm, agg_out, *rest):
        if with_deg:
            deg_out, idx_a, idx_b, rows_a, rows_b, acc_sh, sem_a, sem_b, deg_l = rest
        else:
            idx_a, idx_b, rows_a, rows_b, acc_sh, sem_a, sem_b = rest
        cid = lax.axis_index("c")
        sid = lax.axis_index("s")
        wid = cid * ns + sid

        zero16 = jnp.zeros((16,), jnp.float32)

        def zrow(i, _):
            for v in range(D // 16):
                rows_a[i, pl.ds(v * 16, 16)] = zero16
            return 0

        lax.fori_loop(0, CH, zrow, 0)
        for k in range(rows_per_tile // CH):
            pltpu.sync_copy(rows_a, acc_sh.at[pl.ds(sid * rows_per_tile + k * CH, CH)])

        if with_deg:
            def zdeg(i, _):
                deg_l[pl.ds(i * 16, 16)] = zero16
                return 0
            lax.fori_loop(0, npad // 16, zdeg, 0)

        plsc.subcore_barrier()

        ones16 = jnp.ones((16,), jnp.float32)
        nch = nchunk

        def do_deg(idx_v):
            if with_deg:
                for v in range(CH // 16):
                    idx = idx_v[1, pl.ds(v * 16, 16)]
                    plsc.addupdate_scatter(deg_l, [idx], ones16)

        pltpu.sync_copy(ed_hbm.at[wid, 0], idx_a)
        pltpu.async_copy(y_hbm.at[idx_a.at[0]], rows_a, sem_a)

        def chunk_pair(j, _):
            c1 = 2 * j + 1
            c2 = jnp.minimum(2 * j + 2, nch - 1)
            pltpu.sync_copy(ed_hbm.at[wid, c1], idx_b)
            pltpu.make_async_copy(y_hbm.at[idx_a.at[0]], rows_a, sem_a).wait()
            pltpu.async_copy(y_hbm.at[idx_b.at[0]], rows_b, sem_b)
            pltpu.sync_copy(rows_a, acc_sh.at[idx_a.at[1]], add=True)
            do_deg(idx_a)
            pltpu.sync_copy(ed_hbm.at[wid, c2], idx_a)
            pltpu.make_async_copy(y_hbm.at[idx_b.at[0]], rows_b, sem_b).wait()
            pltpu.async_copy(y_hbm.at[idx_a.at[0]], rows_a, sem_a)
            pltpu.sync_copy(rows_b, acc_sh.at[idx_b.at[1]], add=True)
            do_deg(idx_b)
            return 0

        lax.fori_loop(0, nch // 2, chunk_pair, 0)
        pltpu.make_async_copy(y_hbm.at[idx_a.at[0]], rows_a, sem_a).wait()

        plsc.subcore_barrier()
        pltpu.sync_copy(
            acc_sh.at[pl.ds(sid * rows_per_tile, rows_per_tile)],
            agg_out.at[cid, pl.ds(sid * rows_per_tile, rows_per_tile)],
        )
        if with_deg:
            pltpu.sync_copy(deg_l, deg_out.at[wid])

    return pl.kernel(
        body,
        out_type=out_type,
        mesh=mesh,
        scratch_types=scratch,
        compiler_params=pltpu.CompilerParams(needs_layout_passes=False),
    )


def kernel(x, edge_index, W_self1, W_neigh1, b1, W_self2, W_neigh2, b2):
    n = x.shape[0]
    e = edge_index.shape[1]
    try:
        info = plsc.get_sparse_core_info()
        nc, ns = info.num_cores, info.num_subcores
    except ValueError:
        nc, ns = 2, 16
    nw = nc * ns

    npad = ((n + (ns * CH) - 1) // (ns * CH)) * (ns * CH)
    ep = ((e + (nw * CH) - 1) // (nw * CH)) * (nw * CH)
    nchunk = ep // (nw * CH)

    src = edge_index[0]
    dst = edge_index[1]
    pad = ep - e
    if pad:
        parange = jnp.arange(pad, dtype=jnp.int32)
        src = jnp.concatenate([src, parange % n])
        dst = jnp.concatenate([dst, n + parange % (npad - n)])
    ed = jnp.stack([src.reshape(nw, nchunk, CH), dst.reshape(nw, nchunk, CH)],
                   axis=2)

    b1r = b1.reshape(1, D)
    b2r = b2.reshape(1, D)

    sc_agg_deg = _make_sc_agg(npad, nchunk, True, nc, ns)
    sc_agg = _make_sc_agg(npad, nchunk, False, nc, ns)

    s1, y1 = _tc_dual_matmul(x, W_self1, W_neigh1)
    agg1p, degp = sc_agg_deg(y1, ed)
    degt = degp.T
    s2, y2 = _tc_mid(s1, agg1p, degt, b1r, W_self2, W_neigh2)
    (agg2p,) = sc_agg(y2, ed)
    return _tc_fin(s2, agg2p, degt, b2r)
